# Initial kernel scaffold; baseline (speedup 1.0000x reference)
import jax
import jax.numpy as jnp
from jax.experimental import pallas as pl


def kernel(node_feats, edge_index, W_s, W_pc, T):
    raise NotImplementedError("write your pallas kernel here")


# speedup vs baseline: 3.7883x; 3.7883x over previous
import functools

import jax
import jax.numpy as jnp
from jax import lax
from jax.experimental import pallas as pl
from jax.experimental.pallas import tpu as pltpu
from jax.experimental.pallas import tpu_sc as plsc

N = 10000
E = 320000
D = 128
ITERS = 10

NC = 2
NS = 16
NW = NC * NS

K = 128
CH = 79
EPT = CH * K
E_PAD = NW * EPT
N_PAD = 10112
ROWS_PER_TILE = N_PAD // NS

TC_BLK = 2528
TC_GRID = N_PAD // TC_BLK


def _sc_agg_body(h_hbm, p3_hbm, c3_hbm, out_hbm,
                 pidx, cidx, bufp, bufc, agg, sem_p, sem_c):
    cid = lax.axis_index("c")
    sid = lax.axis_index("s")
    wid = sid * NC + cid

    def _zero_row(r, _):
        for j in range(D // 16):
            bufp[r, pl.ds(j * 16, 16)] = jnp.zeros((16,), jnp.float32)
        return 0

    lax.fori_loop(0, K, _zero_row, 0)
    base = sid * ROWS_PER_TILE
    for t in range(ROWS_PER_TILE // K):
        pltpu.sync_copy(bufp, agg.at[pl.ds(base + t * K, K)])
    rem = ROWS_PER_TILE % K
    if rem:
        pltpu.sync_copy(bufp.at[pl.ds(0, rem)],
                        agg.at[pl.ds(base + (ROWS_PER_TILE // K) * K, rem)])

    plsc.subcore_barrier()

    def _chunk(j, _):
        pltpu.sync_copy(p3_hbm.at[wid].at[j], pidx)
        pltpu.sync_copy(c3_hbm.at[wid].at[j], cidx)
        cp_p = pltpu.async_copy(h_hbm.at[pidx.at[0]], bufp, sem_p)
        cp_c = pltpu.async_copy(h_hbm.at[cidx.at[0]], bufc, sem_c)
        cp_p.wait()
        cp_c.wait()

        def _add_row(r, _):
            for jj in range(D // 16):
                sl = pl.ds(jj * 16, 16)
                bufp[r, sl] = bufp[r, sl] + bufc[r, sl]
            return 0

        lax.fori_loop(0, K, _add_row, 0)
        pltpu.sync_copy(bufp, agg.at[pidx.at[0]], add=True)
        pltpu.sync_copy(bufp, agg.at[cidx.at[0]], add=True)
        return 0

    lax.fori_loop(0, CH, _chunk, 0)

    plsc.subcore_barrier()
    pltpu.sync_copy(agg.at[pl.ds(base, ROWS_PER_TILE)],
                    out_hbm.at[cid].at[pl.ds(base, ROWS_PER_TILE)])


_sc_agg = functools.partial(
    pl.kernel,
    out_type=jax.ShapeDtypeStruct((NC, N_PAD, D), jnp.float32),
    mesh=plsc.VectorSubcoreMesh(core_axis_name="c", subcore_axis_name="s"),
    scratch_types=[
        pltpu.VMEM((1, K), jnp.int32),
        pltpu.VMEM((1, K), jnp.int32),
        pltpu.VMEM((K, D), jnp.float32),
        pltpu.VMEM((K, D), jnp.float32),
        pltpu.VMEM_SHARED((N_PAD, D), jnp.float32),
        pltpu.SemaphoreType.DMA,
        pltpu.SemaphoreType.DMA,
    ],
)(_sc_agg_body)


def _proj_body(x_ref, w_ref, o_ref):
    o_ref[...] = jnp.dot(x_ref[...], w_ref[...],
                         preferred_element_type=jnp.float32,
                         precision=jax.lax.Precision.HIGHEST)


def _proj(x_pad, wsT):
    return pl.pallas_call(
        _proj_body,
        grid=(TC_GRID,),
        in_specs=[
            pl.BlockSpec((TC_BLK, D), lambda i: (i, 0)),
            pl.BlockSpec((D, D), lambda i: (0, 0)),
        ],
        out_specs=pl.BlockSpec((TC_BLK, D), lambda i: (i, 0)),
        out_shape=jax.ShapeDtypeStruct((N_PAD, D), jnp.float32),
    )(x_pad, wsT)


def _update_body(sw_ref, agg_ref, h_ref, acc_ref, w_ref, hn_ref, accn_ref):
    h = h_ref[...]
    agg = agg_ref[0] + agg_ref[1]
    z = jnp.dot(agg, w_ref[...],
                preferred_element_type=jnp.float32,
                precision=jax.lax.Precision.HIGHEST) + h
    step = jnp.maximum(z, 0.0) + h
    hn_ref[...] = step
    accn_ref[...] = acc_ref[...] + sw_ref[0] * step


def _update(agg2, h, acc, wpcT, sw):
    return pl.pallas_call(
        _update_body,
        grid=(TC_GRID,),
        in_specs=[
            pl.BlockSpec(memory_space=pltpu.SMEM),
            pl.BlockSpec((NC, TC_BLK, D), lambda i: (0, i, 0)),
            pl.BlockSpec((TC_BLK, D), lambda i: (i, 0)),
            pl.BlockSpec((TC_BLK, D), lambda i: (i, 0)),
            pl.BlockSpec((D, D), lambda i: (0, 0)),
        ],
        out_specs=[
            pl.BlockSpec((TC_BLK, D), lambda i: (i, 0)),
            pl.BlockSpec((TC_BLK, D), lambda i: (i, 0)),
        ],
        out_shape=[
            jax.ShapeDtypeStruct((N_PAD, D), jnp.float32),
            jax.ShapeDtypeStruct((N_PAD, D), jnp.float32),
        ],
    )(sw, agg2, h, acc, wpcT)


def kernel(node_feats, edge_index, W_s, W_pc, T):
    p = edge_index[0]
    c = edge_index[1]
    pad = E_PAD - E
    dummy = jnp.full((pad,), N, dtype=jnp.int32)
    p3 = jnp.concatenate([p, dummy]).reshape(NW, CH, 1, K)
    c3 = jnp.concatenate([c, dummy]).reshape(NW, CH, 1, K)

    x_pad = jnp.pad(node_feats, ((0, N_PAD - N), (0, 0)))
    wsT = W_s.T
    wpcT = W_pc.T
    sw_all = jax.nn.sigmoid(T - jnp.arange(ITERS, dtype=jnp.float32))

    h = _proj(x_pad, wsT)
    acc = jnp.zeros((N_PAD, D), jnp.float32)
    for i in range(ITERS):
        agg2 = _sc_agg(h, p3, c3)
        h, acc = _update(agg2, h, acc, wpcT, sw_all[i:i + 1])
    return acc[:N]


# speedup vs baseline: 4.2883x; 1.1320x over previous
import functools

import jax
import jax.numpy as jnp
from jax import lax
from jax.experimental import pallas as pl
from jax.experimental.pallas import tpu as pltpu
from jax.experimental.pallas import tpu_sc as plsc

N = 10000
E = 320000
D = 128
DH = 64
ITERS = 10

NC = 2
NS = 16
RING = 4

K = 128
CH = 160
EPT = CH * K
E_PAD = NS * EPT
N_PAD = 10112
ROWS_PER_TILE = N_PAD // NS

TC_BLK = 2528
TC_GRID = N_PAD // TC_BLK


def _sc_agg_body(h_hbm, p3_hbm, c3_hbm, out_hbm, *refs):
    pidx = refs[0:RING]
    cidx = refs[RING:2 * RING]
    bufp = refs[2 * RING:3 * RING]
    bufc = refs[3 * RING:4 * RING]
    agg = refs[4 * RING]
    semgp = refs[4 * RING + 1:5 * RING + 1]
    semgc = refs[5 * RING + 1:6 * RING + 1]
    sems = refs[6 * RING + 1:7 * RING + 1]

    cid = lax.axis_index("c")
    sid = lax.axis_index("s")
    hsrc = h_hbm.at[cid]

    def _zero_row(r, _):
        for jj in range(DH // 16):
            bufp[0][r, pl.ds(jj * 16, 16)] = jnp.zeros((16,), jnp.float32)
        return 0

    lax.fori_loop(0, K, _zero_row, 0)
    base = sid * ROWS_PER_TILE
    for t in range(ROWS_PER_TILE // K):
        pltpu.sync_copy(bufp[0], agg.at[pl.ds(base + t * K, K)])
    rem = ROWS_PER_TILE % K
    if rem:
        pltpu.sync_copy(bufp[0].at[pl.ds(0, rem)],
                        agg.at[pl.ds(base + (ROWS_PER_TILE // K) * K, rem)])

    def load_idx(b, ch):
        pltpu.sync_copy(p3_hbm.at[sid].at[ch], pidx[b])
        pltpu.sync_copy(c3_hbm.at[sid].at[ch], cidx[b])

    def start_gathers(b):
        pltpu.async_copy(hsrc.at[pidx[b].at[0]], bufp[b], semgp[b])
        pltpu.async_copy(hsrc.at[cidx[b].at[0]], bufc[b], semgc[b])

    def wait_gathers(b):
        pltpu.make_async_copy(hsrc.at[pidx[b].at[0]], bufp[b], semgp[b]).wait()
        pltpu.make_async_copy(hsrc.at[cidx[b].at[0]], bufc[b], semgc[b]).wait()

    def start_scatters(b):
        pltpu.async_copy(bufp[b], agg.at[pidx[b].at[0]], sems[b], add=True)
        pltpu.async_copy(bufp[b], agg.at[cidx[b].at[0]], sems[b], add=True)

    def wait_scatters(b):
        pltpu.make_async_copy(bufp[b], agg.at[pidx[b].at[0]], sems[b]).wait()
        pltpu.make_async_copy(bufp[b], agg.at[cidx[b].at[0]], sems[b]).wait()

    load_idx(0, 0)
    start_gathers(0)
    load_idx(1, 1)
    start_gathers(1)
    plsc.subcore_barrier()

    def _step(t, _):
        for b in range(RING):
            j = t * RING + b
            wait_gathers(b)

            def _add_row(r, __, b=b):
                for jj in range(DH // 16):
                    sl = pl.ds(jj * 16, 16)
                    bufp[b][r, sl] = bufp[b][r, sl] + bufc[b][r, sl]
                return 0

            lax.fori_loop(0, K, _add_row, 0)
            start_scatters(b)

            b2 = (b + 2) % RING

            @pl.when(j >= 2)
            def _(b2=b2):
                wait_scatters(b2)

            load_idx(b2, lax.rem(j + 2, CH))
            start_gathers(b2)
        return 0

    lax.fori_loop(0, CH // RING, _step, 0)

    wait_scatters((CH - 2) % RING)
    wait_scatters((CH - 1) % RING)
    wait_gathers(CH % RING)
    wait_gathers((CH + 1) % RING)

    plsc.subcore_barrier()
    pltpu.sync_copy(agg.at[pl.ds(base, ROWS_PER_TILE)],
                    out_hbm.at[cid].at[pl.ds(base, ROWS_PER_TILE)])


_sc_scratch = (
    [pltpu.VMEM((1, K), jnp.int32) for _ in range(2 * RING)]
    + [pltpu.VMEM((K, DH), jnp.float32) for _ in range(2 * RING)]
    + [pltpu.VMEM_SHARED((N_PAD, DH), jnp.float32)]
    + [pltpu.SemaphoreType.DMA for _ in range(3 * RING)]
)

_sc_agg = functools.partial(
    pl.kernel,
    out_type=jax.ShapeDtypeStruct((NC, N_PAD, DH), jnp.float32),
    mesh=plsc.VectorSubcoreMesh(core_axis_name="c", subcore_axis_name="s"),
    scratch_types=_sc_scratch,
    compiler_params=pltpu.CompilerParams(use_tc_tiling_on_sc=False),
)(_sc_agg_body)


def _proj_body(x_ref, w_ref, o_ref):
    z = jnp.dot(x_ref[...], w_ref[...],
                preferred_element_type=jnp.float32,
                precision=jax.lax.Precision.HIGHEST)
    o_ref[0] = z[:, :DH]
    o_ref[1] = z[:, DH:]


def _proj(x_pad, wsT):
    return pl.pallas_call(
        _proj_body,
        grid=(TC_GRID,),
        in_specs=[
            pl.BlockSpec((TC_BLK, D), lambda i: (i, 0)),
            pl.BlockSpec((D, D), lambda i: (0, 0)),
        ],
        out_specs=pl.BlockSpec((NC, TC_BLK, DH), lambda i: (0, i, 0)),
        out_shape=jax.ShapeDtypeStruct((NC, N_PAD, DH), jnp.float32),
    )(x_pad, wsT)


def _update_body(sw_ref, agg_ref, h_ref, acc_ref, w0_ref, w1_ref,
                 hn_ref, accn_ref):
    h0 = h_ref[0]
    h1 = h_ref[1]
    zz = (jnp.dot(agg_ref[0], w0_ref[...],
                  preferred_element_type=jnp.float32,
                  precision=jax.lax.Precision.HIGHEST)
          + jnp.dot(agg_ref[1], w1_ref[...],
                    preferred_element_type=jnp.float32,
                    precision=jax.lax.Precision.HIGHEST))
    step0 = jnp.maximum(zz[:, :DH] + h0, 0.0) + h0
    step1 = jnp.maximum(zz[:, DH:] + h1, 0.0) + h1
    sw = sw_ref[0]
    hn_ref[0] = step0
    hn_ref[1] = step1
    accn_ref[0] = acc_ref[0] + sw * step0
    accn_ref[1] = acc_ref[1] + sw * step1


def _update(agg2, h, acc, w0T, w1T, sw):
    half_spec = pl.BlockSpec((NC, TC_BLK, DH), lambda i: (0, i, 0))
    return pl.pallas_call(
        _update_body,
        grid=(TC_GRID,),
        in_specs=[
            pl.BlockSpec(memory_space=pltpu.SMEM),
            half_spec,
            half_spec,
            half_spec,
            pl.BlockSpec((DH, D), lambda i: (0, 0)),
            pl.BlockSpec((DH, D), lambda i: (0, 0)),
        ],
        out_specs=[half_spec, half_spec],
        out_shape=[
            jax.ShapeDtypeStruct((NC, N_PAD, DH), jnp.float32),
            jax.ShapeDtypeStruct((NC, N_PAD, DH), jnp.float32),
        ],
    )(sw, agg2, h, acc, w0T, w1T)


def kernel(node_feats, edge_index, W_s, W_pc, T):
    p = edge_index[0]
    c = edge_index[1]
    pad = E_PAD - E
    dummy = jnp.full((pad,), N, dtype=jnp.int32)
    p3 = jnp.concatenate([p, dummy]).reshape(NS, CH, 1, K)
    c3 = jnp.concatenate([c, dummy]).reshape(NS, CH, 1, K)

    x_pad = jnp.pad(node_feats, ((0, N_PAD - N), (0, 0)))
    wsT = W_s.T
    wpcT = W_pc.T
    w0T = wpcT[:DH]
    w1T = wpcT[DH:]
    sw_all = jax.nn.sigmoid(T - jnp.arange(ITERS, dtype=jnp.float32))

    h = _proj(x_pad, wsT)
    acc = jnp.zeros((NC, N_PAD, DH), jnp.float32)
    for i in range(ITERS):
        agg2 = _sc_agg(h, p3, c3)
        h, acc = _update(agg2, h, acc, w0T, w1T, sw_all[i:i + 1])
    return jnp.concatenate([acc[0, :N], acc[1, :N]], axis=1)
